# baseline (speedup 1.0000x reference)
import functools

import jax
import jax.numpy as jnp
from jax import lax
from jax.experimental import pallas as pl
from jax.experimental.pallas import tpu as pltpu
from jax.experimental.pallas import tpu_sc as plsc

_NB = 8
_G = 4


def _emb_call(num_cores, num_subcores, R, S, D):
    mesh = plsc.VectorSubcoreMesh(core_axis_name="c", subcore_axis_name="s")
    n_workers = num_cores * num_subcores
    rows_per_w = R // n_workers
    s0 = ((S // 2 + 7) // 8) * 8
    sizes = (s0, S - s0)
    offs = (0, s0)
    assert max(sizes) <= 128 and R == n_workers * rows_per_w
    n_chunks = rows_per_w * 2
    assert n_chunks % _NB == 0 and n_chunks >= 2 * _NB and _NB % 2 == 0 and _G % 2 == 0

    @functools.partial(
        pl.kernel,
        mesh=mesh,
        out_type=jax.ShapeDtypeStruct((R, S, D), jnp.float32),
        compiler_params=pltpu.CompilerParams(use_tc_tiling_on_sc=False),
        scratch_types=[
            pltpu.VMEM((rows_per_w, S), jnp.int32),
            pltpu.VMEM((_NB, s0, D), jnp.float32),
            pltpu.SemaphoreType.DMA((_NB,)),
            pltpu.SemaphoreType.DMA((_NB,)),
        ],
    )
    def emb(x_hbm, table_hbm, out_hbm, idx_v, rows_v, gsems, ssems):
        wid = lax.axis_index("s") * num_cores + lax.axis_index("c")
        row0 = wid * rows_per_w
        pltpu.sync_copy(x_hbm.at[pl.ds(row0, rows_per_w)], idx_v)

        def gather(c, b):
            r, k = c // 2, b % 2
            pltpu.async_copy(
                table_hbm.at[idx_v.at[r].at[pl.ds(offs[k], sizes[k])]],
                rows_v.at[b].at[pl.ds(0, sizes[k])],
                gsems.at[b],
            )

        def scatter(c, b):
            r, k = c // 2, b % 2
            pltpu.async_copy(
                rows_v.at[b].at[pl.ds(0, sizes[k])],
                out_hbm.at[row0 + r].at[pl.ds(offs[k], sizes[k])],
                ssems.at[b],
            )

        def drain_g(b):
            k = b % 2
            pltpu.make_async_copy(
                table_hbm.at[pl.ds(0, sizes[k])],
                rows_v.at[b].at[pl.ds(0, sizes[k])],
                gsems.at[b],
            ).wait()

        def drain_s(b):
            k = b % 2
            pltpu.make_async_copy(
                rows_v.at[b].at[pl.ds(0, sizes[k])],
                out_hbm.at[row0].at[pl.ds(offs[k], sizes[k])],
                ssems.at[b],
            ).wait()

        def step(c, b, drain_scatter, prefetch):
            drain_g(b)
            scatter(c, b)
            if prefetch:
                bp = (b + _G) % _NB
                if drain_scatter:
                    drain_s(bp)
                gather(c + _G, bp)

        for c in range(_G):
            gather(c, c)
        for c in range(_NB):
            step(c, c, drain_scatter=(c + _G >= _NB), prefetch=True)

        def body(i, carry):
            c0 = i * _NB
            for b in range(_NB):
                step(c0 + b, b, drain_scatter=True, prefetch=True)
            return carry

        lax.fori_loop(1, n_chunks // _NB - 1, body, 0)

        c0 = n_chunks - _NB
        for b in range(_NB):
            step(c0 + b, b, drain_scatter=True,
                 prefetch=(c0 + b + _G < n_chunks))
        for b in range(_NB):
            drain_s(b)

    return emb


def kernel(x, table):
    R, S = x.shape
    V, D = table.shape
    info = plsc.get_sparse_core_info()
    return _emb_call(info.num_cores, info.num_subcores, R, S, D)(
        x.astype(jnp.int32), table
    )
